# baseline (speedup 1.0000x reference)
import functools

import jax
import jax.numpy as jnp
from jax import lax
from jax.experimental import pallas as pl
from jax.experimental.pallas import tpu as pltpu
from jax.experimental.pallas import tpu_sc as plsc

VOCAB_P1 = 1000001
EMB = 32
HID = 128
B = 16384
L = 200

NC = 2
NS = 16
NW = NC * NS
NG = B // 128
GPW = NG // NW

V_NEED = VOCAB_P1 - 1

ROWS_BLK = 32768
S_TC = 13 * ROWS_BLK

CHUNK = 256
SC_ROWS = V_NEED - S_TC
SC_HALF = (SC_ROWS // 2 + CHUNK - 1) // CHUNK * CHUNK
N_CHUNKS = (SC_ROWS + CHUNK - 1) // CHUNK
LAST_START = V_NEED - CHUNK
K_ITERS = (N_CHUNKS + NW - 1) // NW
K_PAIRS = (K_ITERS + 1) // 2
SC_OUT = SC_ROWS
TWF = V_NEED

H0_CHUNKS = SC_HALF // CHUNK
H1_CHUNKS = N_CHUNKS - H0_CHUNKS
HK_ITERS = (max(H0_CHUNKS, H1_CHUNKS) + NS - 1) // NS
HK_PAIRS = (HK_ITERS + 1) // 2


def _wc_body(w1_ref, b1_ref, w2_ref, b2_ref, wc_ref):
    w = (w1_ref[...] @ w2_ref[...])[:, 0]
    c = jnp.sum(b1_ref[...] * w2_ref[:, 0]) + b2_ref[0]
    wc_ref[...] = jnp.concatenate([w, jnp.full((16,), c, jnp.float32)])


def _compute_wc(w1, b1, w2, b2):
    return pl.pallas_call(
        _wc_body,
        out_shape=jax.ShapeDtypeStruct((48,), jnp.float32),
    )(w1, b1, w2, b2)


def _tw_tc_body(tab_ref, w1_ref, w2_ref, tw_ref):
    w = (w1_ref[...] @ w2_ref[...])[:, 0]
    tb = tab_ref[...]
    tw_ref[...] = jnp.sum(tb * w[None, :], axis=1).reshape(ROWS_BLK // 128, 128)


def _compute_tw_tc(emb_table, w1, w2):
    return pl.pallas_call(
        _tw_tc_body,
        grid=(S_TC // ROWS_BLK,),
        in_specs=[
            pl.BlockSpec((ROWS_BLK, EMB), lambda i: (i, 0)),
            pl.BlockSpec((EMB, HID), lambda i: (0, 0)),
            pl.BlockSpec((HID, 1), lambda i: (0, 0)),
        ],
        out_specs=pl.BlockSpec((ROWS_BLK // 128, 128), lambda i: (i, 0)),
        out_shape=jax.ShapeDtypeStruct((S_TC // 128, 128), jnp.float32),
    )(emb_table, w1, w2)


@functools.cache
def _make_sc_tw(row0, nchunks, nout):
    mesh = plsc.VectorSubcoreMesh(core_axis_name="c", subcore_axis_name="s",
                                  num_cores=1)

    @functools.partial(
        pl.kernel,
        out_type=jax.ShapeDtypeStruct((nout,), jnp.float32),
        mesh=mesh,
        scratch_types=[
            pltpu.VMEM((CHUNK, EMB), jnp.float32),
            pltpu.VMEM((CHUNK, EMB), jnp.float32),
            pltpu.VMEM((CHUNK,), jnp.float32),
            pltpu.VMEM((48,), jnp.float32),
            pltpu.SemaphoreType.DMA,
            pltpu.SemaphoreType.DMA,
        ],
    )
    def _sc_tw(tab_hbm, wc_hbm, twp_hbm, slab0_v, slab1_v, tw_v, wc_v,
               sem0, sem1):
        wid = lax.axis_index("s")

        pltpu.sync_copy(wc_hbm, wc_v)
        wv0 = wc_v[pl.ds(0, 16)]
        wv1 = wc_v[pl.ds(16, 16)]

        lane = lax.iota(jnp.int32, 16)
        idx_e = (lane & 7) * 2
        idx_o = idx_e + 1
        low = lane < 8
        dnums = lax.GatherDimensionNumbers(
            offset_dims=(), collapsed_slice_dims=(0,), start_index_map=(0,))

        def vperm(v, idx):
            return lax.gather(v, idx[:, None], dnums, (1,),
                              mode=lax.GatherScatterMode.PROMISE_IN_BOUNDS)

        def hadd(a, b):
            return jnp.where(low,
                             vperm(a, idx_e) + vperm(a, idx_o),
                             vperm(b, idx_e) + vperm(b, idx_o))

        def start(k, slab, sem):
            cidx = wid + k * NS

            @pl.when(cidx < nchunks)
            def _():
                cs = pl.multiple_of(
                    jnp.minimum(row0 + cidx * CHUNK, LAST_START), 64)
                pltpu.async_copy(tab_hbm.at[pl.ds(cs, CHUNK)], slab, sem)

        def finish(k, slab, sem):
            cidx = wid + k * NS

            @pl.when(cidx < nchunks)
            def _():
                cs = pl.multiple_of(
                    jnp.minimum(row0 + cidx * CHUNK, LAST_START), 64)
                pltpu.make_async_copy(
                    tab_hbm.at[pl.ds(0, CHUNK)], slab, sem).wait()

                def row_body(r, carry2):
                    base = r * 16
                    ps = [
                        slab[base + j, pl.ds(0, 16)] * wv0
                        + slab[base + j, pl.ds(16, 16)] * wv1
                        for j in range(16)
                    ]
                    qs = [hadd(ps[2 * i], ps[2 * i + 1]) for i in range(8)]
                    rs = [hadd(qs[2 * i], qs[2 * i + 1]) for i in range(4)]
                    ss = [hadd(rs[2 * i], rs[2 * i + 1]) for i in range(2)]
                    tw_v[pl.ds(base, 16)] = hadd(ss[0], ss[1])
                    return carry2

                lax.fori_loop(0, CHUNK // 16, row_body, 0)
                pltpu.sync_copy(tw_v, twp_hbm.at[pl.ds(cs - row0, CHUNK)])

        start(0, slab0_v, sem0)

        def pair_body(m, carry):
            k0 = m * 2
            start(k0 + 1, slab1_v, sem1)
            finish(k0, slab0_v, sem0)
            start(k0 + 2, slab0_v, sem0)
            finish(k0 + 1, slab1_v, sem1)
            return carry

        lax.fori_loop(0, HK_PAIRS, pair_body, 0)

    return _sc_tw


@functools.cache
def _make_sc_fnn():
    mesh = plsc.VectorSubcoreMesh(core_axis_name="c", subcore_axis_name="s")

    @functools.partial(
        pl.kernel,
        out_type=jax.ShapeDtypeStruct((B,), jnp.float32),
        mesh=mesh,
        scratch_types=[
            pltpu.VMEM((L * 128,), jnp.int32),
            pltpu.VMEM((L * 128,), jnp.float32),
            pltpu.VMEM((128,), jnp.float32),
            pltpu.VMEM((16,), jnp.float32),
            pltpu.VMEM_SHARED((TWF,), jnp.float32),
            pltpu.SemaphoreType.DMA,
        ],
    )
    def _sc_fnn(twf_hbm, xt_hbm, wc_hbm, out_hbm, idx_v, vals_v,
                out_v, c_v, tw_sh, sem):
        cid = lax.axis_index("c")
        sid = lax.axis_index("s")
        wid = sid * NC + cid

        @pl.when(sid == 0)
        def _():
            pltpu.sync_copy(twf_hbm, tw_sh)

        pltpu.sync_copy(wc_hbm.at[pl.ds(32, 16)], c_v)
        plsc.subcore_barrier()
        cvec = c_v[...]

        zeros = jnp.zeros((16,), jnp.float32)
        for t in range(GPW):
            g128 = wid * GPW + t
            pltpu.sync_copy(xt_hbm.at[g128], idx_v)
            pltpu.async_copy(tw_sh.at[idx_v], vals_v, sem).wait()

            def body(l, accs):
                base = l * 128
                return tuple(
                    accs[g] + vals_v[pl.ds(base + g * 16, 16)]
                    for g in range(8)
                )

            accs = lax.fori_loop(0, L, body, (zeros,) * 8)
            for g in range(8):
                z = accs[g] * (1.0 / L) + cvec
                out_v[pl.ds(g * 16, 16)] = 1.0 / (1.0 + jnp.exp(-z))
            pltpu.sync_copy(out_v, out_hbm.at[pl.ds(g128 * 128, 128)])

    return _sc_fnn


def kernel(x, emb_table, W1, b1, W2, b2):
    wc = _compute_wc(W1, b1, W2, b2)
    twtc = _compute_tw_tc(emb_table, W1, W2).reshape(S_TC)
    twsc0 = _make_sc_tw(S_TC, H0_CHUNKS, SC_HALF)(emb_table, wc)
    twsc1 = _make_sc_tw(S_TC + SC_HALF, H1_CHUNKS,
                        SC_ROWS - SC_HALF)(emb_table, wc)
    twf = jnp.concatenate([twtc, twsc0, twsc1])
    xt = x.astype(jnp.int32).reshape(NG, 128, L).transpose(0, 2, 1)
    xt = xt.reshape(NG, L * 128)
    out = _make_sc_fnn()(twf, xt, wc)
    return out.reshape(B, 1)


# speedup vs baseline: 1.2011x; 1.2011x over previous
import functools

import jax
import jax.numpy as jnp
from jax import lax
from jax.experimental import pallas as pl
from jax.experimental.pallas import tpu as pltpu
from jax.experimental.pallas import tpu_sc as plsc

VOCAB_P1 = 1000001
EMB = 32
HID = 128
B = 16384
L = 200

NC = 2
NS = 16
NW = NC * NS
NG = B // 128
GPW = NG // NW

V_NEED = VOCAB_P1 - 1

ROWS_BLK = 49152
S_TC = 14 * ROWS_BLK

CHUNK = 256
SC_ROWS = V_NEED - S_TC
N_CHUNKS = (SC_ROWS + CHUNK - 1) // CHUNK
LAST_START = V_NEED - CHUNK
K_ITERS = (N_CHUNKS + NW - 1) // NW
K_PAIRS = (K_ITERS + 1) // 2
SC_OUT = SC_ROWS
TWF = V_NEED


def _wc_body(w1_ref, b1_ref, w2_ref, b2_ref, wc_ref):
    w = (w1_ref[...] @ w2_ref[...])[:, 0]
    c = jnp.sum(b1_ref[...] * w2_ref[:, 0]) + b2_ref[0]
    wc_ref[...] = jnp.concatenate([w, jnp.full((16,), c, jnp.float32)])


def _compute_wc(w1, b1, w2, b2):
    return pl.pallas_call(
        _wc_body,
        out_shape=jax.ShapeDtypeStruct((48,), jnp.float32),
    )(w1, b1, w2, b2)


def _tw_tc_body(tab_ref, w1_ref, w2_ref, tw_ref):
    w = (w1_ref[...] @ w2_ref[...])[:, 0]
    tb = tab_ref[...]
    tw_ref[...] = jnp.sum(tb * w[None, :], axis=1).reshape(ROWS_BLK // 128, 128)


def _compute_tw_tc(emb_table, w1, w2):
    return pl.pallas_call(
        _tw_tc_body,
        grid=(S_TC // ROWS_BLK,),
        in_specs=[
            pl.BlockSpec((ROWS_BLK, EMB), lambda i: (i, 0)),
            pl.BlockSpec((EMB, HID), lambda i: (0, 0)),
            pl.BlockSpec((HID, 1), lambda i: (0, 0)),
        ],
        out_specs=pl.BlockSpec((ROWS_BLK // 128, 128), lambda i: (i, 0)),
        out_shape=jax.ShapeDtypeStruct((S_TC // 128, 128), jnp.float32),
    )(emb_table, w1, w2)


@functools.cache
def _make_sc_tw():
    mesh = plsc.VectorSubcoreMesh(core_axis_name="c", subcore_axis_name="s")

    @functools.partial(
        pl.kernel,
        out_type=jax.ShapeDtypeStruct((SC_OUT,), jnp.float32),
        mesh=mesh,
        scratch_types=[
            pltpu.VMEM((CHUNK, EMB), jnp.float32),
            pltpu.VMEM((CHUNK, EMB), jnp.float32),
            pltpu.VMEM((CHUNK,), jnp.float32),
            pltpu.VMEM((48,), jnp.float32),
            pltpu.SemaphoreType.DMA,
            pltpu.SemaphoreType.DMA,
        ],
    )
    def _sc_tw(tab_hbm, wc_hbm, twp_hbm, slab0_v, slab1_v, tw_v, wc_v,
               sem0, sem1):
        cid = lax.axis_index("c")
        sid = lax.axis_index("s")
        wid = sid * NC + cid

        pltpu.sync_copy(wc_hbm, wc_v)
        wv0 = wc_v[pl.ds(0, 16)]
        wv1 = wc_v[pl.ds(16, 16)]

        lane = lax.iota(jnp.int32, 16)
        idx_e = (lane & 7) * 2
        idx_o = idx_e + 1
        low = lane < 8
        dnums = lax.GatherDimensionNumbers(
            offset_dims=(), collapsed_slice_dims=(0,), start_index_map=(0,))

        def vperm(v, idx):
            return lax.gather(v, idx[:, None], dnums, (1,),
                              mode=lax.GatherScatterMode.PROMISE_IN_BOUNDS)

        def hadd(a, b):
            return jnp.where(low,
                             vperm(a, idx_e) + vperm(a, idx_o),
                             vperm(b, idx_e) + vperm(b, idx_o))

        def start(k, slab, sem):
            cidx = wid + k * NW

            @pl.when(cidx < N_CHUNKS)
            def _():
                cs = pl.multiple_of(
                    jnp.minimum(S_TC + cidx * CHUNK, LAST_START), 64)
                pltpu.async_copy(tab_hbm.at[pl.ds(cs, CHUNK)], slab, sem)

        def finish(k, slab, sem):
            cidx = wid + k * NW

            @pl.when(cidx < N_CHUNKS)
            def _():
                cs = pl.multiple_of(
                    jnp.minimum(S_TC + cidx * CHUNK, LAST_START), 64)
                pltpu.make_async_copy(
                    tab_hbm.at[pl.ds(0, CHUNK)], slab, sem).wait()

                def row_body(r, carry2):
                    base = r * 16
                    ps = [
                        slab[base + j, pl.ds(0, 16)] * wv0
                        + slab[base + j, pl.ds(16, 16)] * wv1
                        for j in range(16)
                    ]
                    qs = [hadd(ps[2 * i], ps[2 * i + 1]) for i in range(8)]
                    rs = [hadd(qs[2 * i], qs[2 * i + 1]) for i in range(4)]
                    ss = [hadd(rs[2 * i], rs[2 * i + 1]) for i in range(2)]
                    tw_v[pl.ds(base, 16)] = hadd(ss[0], ss[1])
                    return carry2

                lax.fori_loop(0, CHUNK // 16, row_body, 0)
                pltpu.sync_copy(tw_v, twp_hbm.at[pl.ds(cs - S_TC, CHUNK)])

        start(0, slab0_v, sem0)

        def pair_body(m, carry):
            k0 = m * 2
            start(k0 + 1, slab1_v, sem1)
            finish(k0, slab0_v, sem0)
            start(k0 + 2, slab0_v, sem0)
            finish(k0 + 1, slab1_v, sem1)
            return carry

        lax.fori_loop(0, K_PAIRS, pair_body, 0)

    return _sc_tw


@functools.cache
def _make_sc_fnn():
    mesh = plsc.VectorSubcoreMesh(core_axis_name="c", subcore_axis_name="s")

    @functools.partial(
        pl.kernel,
        out_type=jax.ShapeDtypeStruct((B,), jnp.float32),
        mesh=mesh,
        scratch_types=[
            pltpu.VMEM((L * 128,), jnp.int32),
            pltpu.VMEM((L * 128,), jnp.float32),
            pltpu.VMEM((128,), jnp.float32),
            pltpu.VMEM((16,), jnp.float32),
            pltpu.VMEM_SHARED((TWF,), jnp.float32),
            pltpu.SemaphoreType.DMA,
        ],
    )
    def _sc_fnn(twf_hbm, xt_hbm, wc_hbm, out_hbm, idx_v, vals_v,
                out_v, c_v, tw_sh, sem):
        cid = lax.axis_index("c")
        sid = lax.axis_index("s")
        wid = sid * NC + cid

        @pl.when(sid == 0)
        def _():
            pltpu.sync_copy(twf_hbm, tw_sh)

        pltpu.sync_copy(wc_hbm.at[pl.ds(32, 16)], c_v)
        plsc.subcore_barrier()
        cvec = c_v[...]

        zeros = jnp.zeros((16,), jnp.float32)
        for t in range(GPW):
            g128 = wid * GPW + t
            pltpu.sync_copy(xt_hbm.at[g128], idx_v)
            pltpu.async_copy(tw_sh.at[idx_v], vals_v, sem).wait()

            def body(l, accs):
                base = l * 128
                return tuple(
                    accs[g] + vals_v[pl.ds(base + g * 16, 16)]
                    for g in range(8)
                )

            accs = lax.fori_loop(0, L, body, (zeros,) * 8)
            for g in range(8):
                z = accs[g] * (1.0 / L) + cvec
                out_v[pl.ds(g * 16, 16)] = 1.0 / (1.0 + jnp.exp(-z))
            pltpu.sync_copy(out_v, out_hbm.at[pl.ds(g128 * 128, 128)])

    return _sc_fnn


def kernel(x, emb_table, W1, b1, W2, b2):
    wc = _compute_wc(W1, b1, W2, b2)
    twtc = _compute_tw_tc(emb_table, W1, W2).reshape(S_TC)
    twsc = _make_sc_tw()(emb_table, wc)
    twf = jnp.concatenate([twtc, twsc])
    xt = x.astype(jnp.int32).reshape(NG, 128, L).transpose(0, 2, 1)
    xt = xt.reshape(NG, L * 128)
    out = _make_sc_fnn()(twf, xt, wc)
    return out.reshape(B, 1)


# speedup vs baseline: 1.2020x; 1.0008x over previous
import functools

import jax
import jax.numpy as jnp
from jax import lax
from jax.experimental import pallas as pl
from jax.experimental.pallas import tpu as pltpu
from jax.experimental.pallas import tpu_sc as plsc

VOCAB_P1 = 1000001
EMB = 32
HID = 128
B = 16384
L = 200

NC = 2
NS = 16
NW = NC * NS
NG = B // 128
GPW = NG // NW

V_NEED = VOCAB_P1 - 1

ROWS_BLK = 49152
S_TC = 14 * ROWS_BLK

CHUNK = 256
SC_ROWS = V_NEED - S_TC
N_CHUNKS = (SC_ROWS + CHUNK - 1) // CHUNK
LAST_START = V_NEED - CHUNK
K_ITERS = (N_CHUNKS + NW - 1) // NW
K_PAIRS = (K_ITERS + 1) // 2
SC_OUT = SC_ROWS
TWF = V_NEED


def _wc_body(w1_ref, b1_ref, w2_ref, b2_ref, wc_ref):
    w = (w1_ref[...] @ w2_ref[...])[:, 0]
    c = jnp.sum(b1_ref[...] * w2_ref[:, 0]) + b2_ref[0]
    wc_ref[...] = jnp.concatenate([w, jnp.full((16,), c, jnp.float32)])


def _compute_wc(w1, b1, w2, b2):
    return pl.pallas_call(
        _wc_body,
        out_shape=jax.ShapeDtypeStruct((48,), jnp.float32),
    )(w1, b1, w2, b2)


TC_BLK = 24576
TC_HALF_BLKS = S_TC // (2 * TC_BLK)


def _tw_tc_body(ta_ref, tb_ref, w1_ref, w2_ref, twa_ref, twb_ref):
    w = (w1_ref[...] @ w2_ref[...])[:, 0]
    twa_ref[...] = jnp.sum(ta_ref[...] * w[None, :],
                           axis=1).reshape(TC_BLK // 128, 128)
    twb_ref[...] = jnp.sum(tb_ref[...] * w[None, :],
                           axis=1).reshape(TC_BLK // 128, 128)


def _compute_tw_tc(emb_table, w1, w2):
    twa, twb = pl.pallas_call(
        _tw_tc_body,
        grid=(TC_HALF_BLKS,),
        in_specs=[
            pl.BlockSpec((TC_BLK, EMB), lambda i: (i, 0)),
            pl.BlockSpec((TC_BLK, EMB), lambda i: (i + TC_HALF_BLKS, 0)),
            pl.BlockSpec((EMB, HID), lambda i: (0, 0)),
            pl.BlockSpec((HID, 1), lambda i: (0, 0)),
        ],
        out_specs=[
            pl.BlockSpec((TC_BLK // 128, 128), lambda i: (i, 0)),
            pl.BlockSpec((TC_BLK // 128, 128), lambda i: (i, 0)),
        ],
        out_shape=[
            jax.ShapeDtypeStruct((S_TC // 256, 128), jnp.float32),
            jax.ShapeDtypeStruct((S_TC // 256, 128), jnp.float32),
        ],
    )(emb_table, emb_table, w1, w2)
    return jnp.concatenate([twa.reshape(S_TC // 2), twb.reshape(S_TC // 2)])


@functools.cache
def _make_sc_tw():
    mesh = plsc.VectorSubcoreMesh(core_axis_name="c", subcore_axis_name="s")

    @functools.partial(
        pl.kernel,
        out_type=jax.ShapeDtypeStruct((SC_OUT,), jnp.float32),
        mesh=mesh,
        scratch_types=[
            pltpu.VMEM((CHUNK, EMB), jnp.float32),
            pltpu.VMEM((CHUNK, EMB), jnp.float32),
            pltpu.VMEM((CHUNK,), jnp.float32),
            pltpu.VMEM((48,), jnp.float32),
            pltpu.SemaphoreType.DMA,
            pltpu.SemaphoreType.DMA,
        ],
    )
    def _sc_tw(tab_hbm, wc_hbm, twp_hbm, slab0_v, slab1_v, tw_v, wc_v,
               sem0, sem1):
        cid = lax.axis_index("c")
        sid = lax.axis_index("s")
        wid = sid * NC + cid

        pltpu.sync_copy(wc_hbm, wc_v)
        wv0 = wc_v[pl.ds(0, 16)]
        wv1 = wc_v[pl.ds(16, 16)]

        lane = lax.iota(jnp.int32, 16)
        idx_e = (lane & 7) * 2
        idx_o = idx_e + 1
        low = lane < 8
        dnums = lax.GatherDimensionNumbers(
            offset_dims=(), collapsed_slice_dims=(0,), start_index_map=(0,))

        def vperm(v, idx):
            return lax.gather(v, idx[:, None], dnums, (1,),
                              mode=lax.GatherScatterMode.PROMISE_IN_BOUNDS)

        def hadd(a, b):
            return jnp.where(low,
                             vperm(a, idx_e) + vperm(a, idx_o),
                             vperm(b, idx_e) + vperm(b, idx_o))

        def start(k, slab, sem):
            cidx = wid + k * NW

            @pl.when(cidx < N_CHUNKS)
            def _():
                cs = pl.multiple_of(
                    jnp.minimum(S_TC + cidx * CHUNK, LAST_START), 64)
                pltpu.async_copy(tab_hbm.at[pl.ds(cs, CHUNK)], slab, sem)

        def finish(k, slab, sem):
            cidx = wid + k * NW

            @pl.when(cidx < N_CHUNKS)
            def _():
                cs = pl.multiple_of(
                    jnp.minimum(S_TC + cidx * CHUNK, LAST_START), 64)
                pltpu.make_async_copy(
                    tab_hbm.at[pl.ds(0, CHUNK)], slab, sem).wait()

                def row_body(r, carry2):
                    base = r * 16
                    ps = [
                        slab[base + j, pl.ds(0, 16)] * wv0
                        + slab[base + j, pl.ds(16, 16)] * wv1
                        for j in range(16)
                    ]
                    qs = [hadd(ps[2 * i], ps[2 * i + 1]) for i in range(8)]
                    rs = [hadd(qs[2 * i], qs[2 * i + 1]) for i in range(4)]
                    ss = [hadd(rs[2 * i], rs[2 * i + 1]) for i in range(2)]
                    tw_v[pl.ds(base, 16)] = hadd(ss[0], ss[1])
                    return carry2

                lax.fori_loop(0, CHUNK // 16, row_body, 0)
                pltpu.sync_copy(tw_v, twp_hbm.at[pl.ds(cs - S_TC, CHUNK)])

        start(0, slab0_v, sem0)

        def pair_body(m, carry):
            k0 = m * 2
            start(k0 + 1, slab1_v, sem1)
            finish(k0, slab0_v, sem0)
            start(k0 + 2, slab0_v, sem0)
            finish(k0 + 1, slab1_v, sem1)
            return carry

        lax.fori_loop(0, K_PAIRS, pair_body, 0)

    return _sc_tw


@functools.cache
def _make_sc_fnn():
    mesh = plsc.VectorSubcoreMesh(core_axis_name="c", subcore_axis_name="s")

    @functools.partial(
        pl.kernel,
        out_type=jax.ShapeDtypeStruct((B,), jnp.float32),
        mesh=mesh,
        scratch_types=[
            pltpu.VMEM((L * 128,), jnp.int32),
            pltpu.VMEM((L * 128,), jnp.float32),
            pltpu.VMEM((128,), jnp.float32),
            pltpu.VMEM((16,), jnp.float32),
            pltpu.VMEM_SHARED((TWF,), jnp.float32),
            pltpu.SemaphoreType.DMA,
        ],
    )
    def _sc_fnn(twf_hbm, xt_hbm, wc_hbm, out_hbm, idx_v, vals_v,
                out_v, c_v, tw_sh, sem):
        cid = lax.axis_index("c")
        sid = lax.axis_index("s")
        wid = sid * NC + cid

        @pl.when(sid == 0)
        def _():
            pltpu.sync_copy(twf_hbm, tw_sh)

        pltpu.sync_copy(wc_hbm.at[pl.ds(32, 16)], c_v)
        plsc.subcore_barrier()
        cvec = c_v[...]

        zeros = jnp.zeros((16,), jnp.float32)
        for t in range(GPW):
            g128 = wid * GPW + t
            pltpu.sync_copy(xt_hbm.at[g128], idx_v)
            pltpu.async_copy(tw_sh.at[idx_v], vals_v, sem).wait()

            def body(l, accs):
                base = l * 128
                return tuple(
                    accs[g] + vals_v[pl.ds(base + g * 16, 16)]
                    for g in range(8)
                )

            accs = lax.fori_loop(0, L, body, (zeros,) * 8)
            for g in range(8):
                z = accs[g] * (1.0 / L) + cvec
                out_v[pl.ds(g * 16, 16)] = 1.0 / (1.0 + jnp.exp(-z))
            pltpu.sync_copy(out_v, out_hbm.at[pl.ds(g128 * 128, 128)])

    return _sc_fnn


def kernel(x, emb_table, W1, b1, W2, b2):
    wc = _compute_wc(W1, b1, W2, b2)
    twtc = _compute_tw_tc(emb_table, W1, W2)
    twsc = _make_sc_tw()(emb_table, wc)
    twf = jnp.concatenate([twtc, twsc])
    xt = x.astype(jnp.int32).reshape(NG, 128, L).transpose(0, 2, 1)
    xt = xt.reshape(NG, L * 128)
    out = _make_sc_fnn()(twf, xt, wc)
    return out.reshape(B, 1)


# speedup vs baseline: 1.2049x; 1.0024x over previous
import functools

import jax
import jax.numpy as jnp
from jax import lax
from jax.experimental import pallas as pl
from jax.experimental.pallas import tpu as pltpu
from jax.experimental.pallas import tpu_sc as plsc

VOCAB_P1 = 1000001
EMB = 32
HID = 128
B = 16384
L = 200

NC = 2
NS = 16
NW = NC * NS
NG = B // 128
GPW = NG // NW

V_NEED = VOCAB_P1 - 1

ROWS_BLK = 49152
S_TC = 13 * ROWS_BLK

CHUNK = 256
SC_ROWS = V_NEED - S_TC
N_CHUNKS = (SC_ROWS + CHUNK - 1) // CHUNK
LAST_START = V_NEED - CHUNK
K_ITERS = (N_CHUNKS + NW - 1) // NW
K_PAIRS = (K_ITERS + 1) // 2
SC_OUT = (SC_ROWS + 127) // 128 * 128
TWF = S_TC + SC_OUT


def _wc_body(w1_ref, b1_ref, w2_ref, b2_ref, wc_ref):
    w = (w1_ref[...] @ w2_ref[...])[:, 0]
    c = jnp.sum(b1_ref[...] * w2_ref[:, 0]) + b2_ref[0]
    wc_ref[...] = jnp.concatenate([w, jnp.full((16,), c, jnp.float32)])


def _compute_wc(w1, b1, w2, b2):
    return pl.pallas_call(
        _wc_body,
        out_shape=jax.ShapeDtypeStruct((48,), jnp.float32),
    )(w1, b1, w2, b2)


TC_BLK = 24576
TC_HALF_BLKS = S_TC // (2 * TC_BLK)


def _tw_tc_body(ta_ref, tb_ref, w1_ref, w2_ref, twa_ref, twb_ref):
    w = (w1_ref[...] @ w2_ref[...])[:, 0]
    twa_ref[...] = jnp.sum(ta_ref[...] * w[None, :],
                           axis=1).reshape(TC_BLK // 128, 128)
    twb_ref[...] = jnp.sum(tb_ref[...] * w[None, :],
                           axis=1).reshape(TC_BLK // 128, 128)


def _compute_tw_tc(emb_table, w1, w2):
    twa, twb = pl.pallas_call(
        _tw_tc_body,
        grid=(TC_HALF_BLKS,),
        in_specs=[
            pl.BlockSpec((TC_BLK, EMB), lambda i: (i, 0)),
            pl.BlockSpec((TC_BLK, EMB), lambda i: (i + TC_HALF_BLKS, 0)),
            pl.BlockSpec((EMB, HID), lambda i: (0, 0)),
            pl.BlockSpec((HID, 1), lambda i: (0, 0)),
        ],
        out_specs=[
            pl.BlockSpec((TC_BLK // 128, 128), lambda i: (i, 0)),
            pl.BlockSpec((TC_BLK // 128, 128), lambda i: (i, 0)),
        ],
        out_shape=[
            jax.ShapeDtypeStruct((S_TC // 256, 128), jnp.float32),
            jax.ShapeDtypeStruct((S_TC // 256, 128), jnp.float32),
        ],
    )(emb_table, emb_table, w1, w2)
    return jnp.concatenate([twa.reshape(S_TC // 2), twb.reshape(S_TC // 2)])


@functools.cache
def _make_sc_tw():
    mesh = plsc.VectorSubcoreMesh(core_axis_name="c", subcore_axis_name="s")

    @functools.partial(
        pl.kernel,
        out_type=jax.ShapeDtypeStruct((SC_OUT,), jnp.float32),
        mesh=mesh,
        scratch_types=[
            pltpu.VMEM((CHUNK, EMB), jnp.float32),
            pltpu.VMEM((CHUNK, EMB), jnp.float32),
            pltpu.VMEM((CHUNK,), jnp.float32),
            pltpu.VMEM((48,), jnp.float32),
            pltpu.SemaphoreType.DMA,
            pltpu.SemaphoreType.DMA,
        ],
    )
    def _sc_tw(tab_hbm, wc_hbm, twp_hbm, slab0_v, slab1_v, tw_v, wc_v,
               sem0, sem1):
        cid = lax.axis_index("c")
        sid = lax.axis_index("s")
        wid = sid * NC + cid

        pltpu.sync_copy(wc_hbm, wc_v)
        wv0 = wc_v[pl.ds(0, 16)]
        wv1 = wc_v[pl.ds(16, 16)]

        lane = lax.iota(jnp.int32, 16)
        idx_e = (lane & 7) * 2
        idx_o = idx_e + 1
        low = lane < 8
        dnums = lax.GatherDimensionNumbers(
            offset_dims=(), collapsed_slice_dims=(0,), start_index_map=(0,))

        def vperm(v, idx):
            return lax.gather(v, idx[:, None], dnums, (1,),
                              mode=lax.GatherScatterMode.PROMISE_IN_BOUNDS)

        def hadd(a, b):
            return jnp.where(low,
                             vperm(a, idx_e) + vperm(a, idx_o),
                             vperm(b, idx_e) + vperm(b, idx_o))

        def start(k, slab, sem):
            cidx = wid + k * NW

            @pl.when(cidx < N_CHUNKS)
            def _():
                cs = pl.multiple_of(
                    jnp.minimum(S_TC + cidx * CHUNK, LAST_START), 64)
                pltpu.async_copy(tab_hbm.at[pl.ds(cs, CHUNK)], slab, sem)

        def finish(k, slab, sem):
            cidx = wid + k * NW

            @pl.when(cidx < N_CHUNKS)
            def _():
                cs = pl.multiple_of(
                    jnp.minimum(S_TC + cidx * CHUNK, LAST_START), 64)
                pltpu.make_async_copy(
                    tab_hbm.at[pl.ds(0, CHUNK)], slab, sem).wait()

                def row_body(r, carry2):
                    base = r * 16
                    ps = [
                        slab[base + j, pl.ds(0, 16)] * wv0
                        + slab[base + j, pl.ds(16, 16)] * wv1
                        for j in range(16)
                    ]
                    qs = [hadd(ps[2 * i], ps[2 * i + 1]) for i in range(8)]
                    rs = [hadd(qs[2 * i], qs[2 * i + 1]) for i in range(4)]
                    ss = [hadd(rs[2 * i], rs[2 * i + 1]) for i in range(2)]
                    tw_v[pl.ds(base, 16)] = hadd(ss[0], ss[1])
                    return carry2

                lax.fori_loop(0, CHUNK // 16, row_body, 0)
                pltpu.sync_copy(tw_v, twp_hbm.at[pl.ds(cs - S_TC, CHUNK)])

        start(0, slab0_v, sem0)

        def pair_body(m, carry):
            k0 = m * 2
            start(k0 + 1, slab1_v, sem1)
            finish(k0, slab0_v, sem0)
            start(k0 + 2, slab0_v, sem0)
            finish(k0 + 1, slab1_v, sem1)
            return carry

        lax.fori_loop(0, K_PAIRS, pair_body, 0)

    return _sc_tw


@functools.cache
def _make_sc_fnn():
    mesh = plsc.VectorSubcoreMesh(core_axis_name="c", subcore_axis_name="s")

    @functools.partial(
        pl.kernel,
        out_type=jax.ShapeDtypeStruct((B,), jnp.float32),
        mesh=mesh,
        scratch_types=[
            pltpu.VMEM((L * 128,), jnp.int32),
            pltpu.VMEM((L * 128,), jnp.float32),
            pltpu.VMEM((128,), jnp.float32),
            pltpu.VMEM((16,), jnp.float32),
            pltpu.VMEM_SHARED((TWF,), jnp.float32),
            pltpu.SemaphoreType.DMA,
        ],
    )
    def _sc_fnn(twtc_hbm, twsc_hbm, xt_hbm, wc_hbm, out_hbm, idx_v, vals_v,
                out_v, c_v, tw_sh, sem):
        cid = lax.axis_index("c")
        sid = lax.axis_index("s")
        wid = sid * NC + cid

        @pl.when(sid == 0)
        def _():
            pltpu.sync_copy(twtc_hbm, tw_sh.at[pl.ds(0, S_TC)])
            pltpu.sync_copy(twsc_hbm, tw_sh.at[pl.ds(S_TC, SC_OUT)])

        pltpu.sync_copy(wc_hbm.at[pl.ds(32, 16)], c_v)
        plsc.subcore_barrier()
        cvec = c_v[...]

        zeros = jnp.zeros((16,), jnp.float32)
        for t in range(GPW):
            g128 = wid * GPW + t
            pltpu.sync_copy(xt_hbm.at[g128], idx_v)
            pltpu.async_copy(tw_sh.at[idx_v], vals_v, sem).wait()

            def body(l, accs):
                base = l * 128
                return tuple(
                    accs[g] + vals_v[pl.ds(base + g * 16, 16)]
                    for g in range(8)
                )

            accs = lax.fori_loop(0, L, body, (zeros,) * 8)
            for g in range(8):
                z = accs[g] * (1.0 / L) + cvec
                out_v[pl.ds(g * 16, 16)] = 1.0 / (1.0 + jnp.exp(-z))
            pltpu.sync_copy(out_v, out_hbm.at[pl.ds(g128 * 128, 128)])

    return _sc_fnn


def kernel(x, emb_table, W1, b1, W2, b2):
    wc = _compute_wc(W1, b1, W2, b2)
    twtc = _compute_tw_tc(emb_table, W1, W2)
    twsc = _make_sc_tw()(emb_table, wc)
    xt = x.astype(jnp.int32).reshape(NG, 128, L).transpose(0, 2, 1)
    xt = xt.reshape(NG, L * 128)
    out = _make_sc_fnn()(twtc, twsc, xt, wc)
    return out.reshape(B, 1)
